# baseline (speedup 1.0000x reference)
import numpy as np
import jax
import jax.numpy as jnp
from jax.experimental import pallas as pl
from jax.experimental.pallas import tpu as pltpu

_SIGMA1 = 0.02
_K = 128
_TMIN = 1e-10
_LEAKY = 0.01
_LN_S1 = float(np.log(_SIGMA1))

_B_BLK = 128
_C_BLK = 128
_NOCT = _B_BLK // 8


def _fused_kernel(x_ref, t_ref, noise_ref, W1_ref, b1_ref, W2_ref, b2_ref,
                  out_ref, mu_scr, h_scr, o_scr, z0_scr, ds_scr, po_scr):
    D = x_ref.shape[1]
    t = t_ref[...]
    gamma = 1.0 - jnp.exp((2.0 * _LN_S1) * t)
    mu_coef = gamma * (1.0 - gamma)
    mu_scr[...] = gamma * x_ref[...] + mu_coef * noise_ref[...]

    h = jnp.dot(mu_scr[...], W1_ref[:D, :], preferred_element_type=jnp.float32)
    h = h + t * W1_ref[D:D + 1, :] + b1_ref[...]
    h_scr[...] = jnp.where(h >= 0, h, _LEAKY * h).astype(jnp.bfloat16)

    o_scr[...] = jnp.dot(h_scr[...], W2_ref[...], preferred_element_type=jnp.float32)

    inv_gamma = 1.0 / gamma
    r = jnp.sqrt((1.0 - gamma) * inv_gamma)
    low_t = t < _TMIN

    for c in range(D // _C_BLK):
        lo, hi = c * _C_BLK, (c + 1) * _C_BLK
        mu_eps = o_scr[:, lo:hi] + b2_ref[:, lo:hi]
        ln_sig = o_scr[:, D + lo:D + hi] + b2_ref[:, D + lo:D + hi]
        mu_x = mu_scr[:, lo:hi] * inv_gamma - r * mu_eps
        sigma_x = r * jnp.exp(ln_sig)
        mu_x = jnp.where(low_t, 0.0, mu_x)
        sigma_x = jnp.where(low_t, 1.0, sigma_x)
        inv = jnp.minimum(1.0 / (sigma_x * jnp.float32(np.sqrt(2.0))), 1e30)
        ds_scr[:, :, lo:hi] = ((2.0 / _K) * inv).reshape(_NOCT, 8, _C_BLK)
        z0_scr[:, :, lo:hi] = ((-1.0 - mu_x) * inv).reshape(_NOCT, 8, _C_BLK)

    sub_iota = jax.lax.broadcasted_iota(
        jnp.int32, (8, _C_BLK), 0).astype(jnp.float32)
    for c in range(D // _C_BLK):
        lo, hi = c * _C_BLK, (c + 1) * _C_BLK

        def oct_body(r8, carry):
            z0o = z0_scr[pl.ds(r8, 1), :, lo:hi].reshape(8, _C_BLK)
            dso = ds_scr[pl.ds(r8, 1), :, lo:hi].reshape(8, _C_BLK)
            t_rows = []
            for rr in range(8):
                z0b = jnp.broadcast_to(z0o[rr:rr + 1, :], (8, _C_BLK))
                dsb = jnp.broadcast_to(dso[rr:rr + 1, :], (8, _C_BLK))
                ds8 = 8.0 * dsb
                zv = sub_iota * dsb + z0b
                es = []
                for v in range(8):
                    znext = zv + ds8
                    zpair = jnp.concatenate([zv, znext], axis=0)
                    es.append(jax.lax.erf(zpair.astype(jnp.bfloat16)))
                    zv = znext + ds8
                acc = ((es[0] + es[1]).astype(jnp.float32)
                       + (es[2] + es[3]).astype(jnp.float32)
                       + (es[4] + es[5]).astype(jnp.float32)
                       + (es[6] + es[7]).astype(jnp.float32))
                t_rows.append(jnp.sum(acc, axis=0, keepdims=True))
            po_scr[pl.ds(r8, 1), :, lo:hi] = jnp.concatenate(
                t_rows, axis=0).reshape(1, 8, _C_BLK)
            return carry

        jax.lax.fori_loop(0, _NOCT, oct_body, 0)

    w_row = jnp.exp((-2.0 * _LN_S1) * t)
    acc2 = jnp.zeros((_B_BLK, _C_BLK), jnp.float32)
    for c in range(D // _C_BLK):
        lo, hi = c * _C_BLK, (c + 1) * _C_BLK
        bigt = po_scr[:, :, lo:hi].reshape(_B_BLK, _C_BLK)
        z0 = z0_scr[:, :, lo:hi].reshape(_B_BLK, _C_BLK)
        ds = ds_scr[:, :, lo:hi].reshape(_B_BLK, _C_BLK)
        e0 = jax.lax.erf(z0)
        e127 = jax.lax.erf(z0 + 127.0 * ds)
        pO = ((-127.0 / 256.0)
              + (1.0 / 128.0) * (e0 - bigt)
              + (127.0 / 256.0) * e127)
        d = x_ref[:, lo:hi] - pO
        acc2 = acc2 + d * d
    out_ref[...] = jnp.sum(w_row * acc2, axis=0, keepdims=True).reshape(
        1, 1, _C_BLK)


def kernel(x, t, noise, W1, b1, W2, b2):
    B, D = x.shape
    H = W1.shape[1]
    nb = B // _B_BLK
    grid = (nb,)
    parts = pl.pallas_call(
        _fused_kernel,
        grid=grid,
        in_specs=[
            pl.BlockSpec((_B_BLK, D), lambda i: (i, 0)),
            pl.BlockSpec((_B_BLK, 1), lambda i: (i, 0)),
            pl.BlockSpec((_B_BLK, D), lambda i: (i, 0)),
            pl.BlockSpec((D + 1, H), lambda i: (0, 0)),
            pl.BlockSpec((1, H), lambda i: (0, 0)),
            pl.BlockSpec((H, 2 * D), lambda i: (0, 0)),
            pl.BlockSpec((1, 2 * D), lambda i: (0, 0)),
        ],
        out_specs=pl.BlockSpec((1, 1, _C_BLK), lambda i: (i, 0, 0)),
        out_shape=jax.ShapeDtypeStruct((nb, 1, _C_BLK), jnp.float32),
        scratch_shapes=[
            pltpu.VMEM((_B_BLK, D), jnp.float32),
            pltpu.VMEM((_B_BLK, H), jnp.bfloat16),
            pltpu.VMEM((_B_BLK, 2 * D), jnp.float32),
            pltpu.VMEM((_NOCT, 8, D), jnp.float32),
            pltpu.VMEM((_NOCT, 8, D), jnp.float32),
            pltpu.VMEM((_NOCT, 8, D), jnp.float32),
        ],
        compiler_params=pltpu.CompilerParams(
            dimension_semantics=("arbitrary",),
            vmem_limit_bytes=100 * 1024 * 1024,
        ),
    )(x, t, noise, W1, b1.reshape(1, H), W2, b2.reshape(1, 2 * D))
    return (-_LN_S1 / (B * D)) * jnp.sum(parts)


# speedup vs baseline: 1.0790x; 1.0790x over previous
import numpy as np
import jax
import jax.numpy as jnp
from jax.experimental import pallas as pl
from jax.experimental.pallas import tpu as pltpu

_SIGMA1 = 0.02
_K = 128
_TMIN = 1e-10
_LEAKY = 0.01
_LN_S1 = float(np.log(_SIGMA1))

_B_BLK = 128
_C_BLK = 128
_NOCT = _B_BLK // 8


def _fused_kernel(x_ref, t_ref, noise_ref, W1_ref, b1_ref, W2_ref, b2_ref,
                  out_ref, mu_scr, h_scr, o_scr, z0_scr, ds_scr, po_scr):
    D = x_ref.shape[1]
    t = t_ref[...]
    gamma = 1.0 - jnp.exp((2.0 * _LN_S1) * t)
    mu_coef = gamma * (1.0 - gamma)
    mu_scr[...] = gamma * x_ref[...] + mu_coef * noise_ref[...]

    h = jnp.dot(mu_scr[...], W1_ref[:D, :], preferred_element_type=jnp.float32)
    h = h + t * W1_ref[D:D + 1, :] + b1_ref[...]
    h_scr[...] = jnp.where(h >= 0, h, _LEAKY * h).astype(jnp.bfloat16)

    o_scr[...] = jnp.dot(h_scr[...], W2_ref[...], preferred_element_type=jnp.float32)

    inv_gamma = 1.0 / gamma
    r = jnp.sqrt((1.0 - gamma) * inv_gamma)
    low_t = t < _TMIN

    for c in range(D // _C_BLK):
        lo, hi = c * _C_BLK, (c + 1) * _C_BLK
        mu_eps = o_scr[:, lo:hi] + b2_ref[:, lo:hi]
        ln_sig = o_scr[:, D + lo:D + hi] + b2_ref[:, D + lo:D + hi]
        mu_x = mu_scr[:, lo:hi] * inv_gamma - r * mu_eps
        sigma_x = r * jnp.exp(ln_sig)
        mu_x = jnp.where(low_t, 0.0, mu_x)
        sigma_x = jnp.where(low_t, 1.0, sigma_x)
        inv = jnp.minimum(1.0 / (sigma_x * jnp.float32(np.sqrt(2.0))), 1e30)
        ds_scr[:, :, lo:hi] = ((2.0 / _K) * inv).reshape(_NOCT, 8, _C_BLK)
        z0_scr[:, :, lo:hi] = ((-1.0 - mu_x) * inv).reshape(_NOCT, 8, _C_BLK)

    sub_iota = jax.lax.broadcasted_iota(
        jnp.int32, (8, _C_BLK), 0).astype(jnp.float32)
    for c in range(D // _C_BLK):
        lo, hi = c * _C_BLK, (c + 1) * _C_BLK

        def oct_body(r8, carry):
            z0o = z0_scr[pl.ds(r8, 1), :, lo:hi].reshape(8, _C_BLK)
            dso = ds_scr[pl.ds(r8, 1), :, lo:hi].reshape(8, _C_BLK)
            t_rows = []
            for rr in range(8):
                z0b = jnp.broadcast_to(z0o[rr:rr + 1, :], (8, _C_BLK))
                dsb = jnp.broadcast_to(dso[rr:rr + 1, :], (8, _C_BLK))
                ds8 = 8.0 * dsb
                zv = sub_iota * dsb + z0b
                es = []
                for v in range(8):
                    znext = zv + ds8
                    zpair = jnp.concatenate([zv, znext], axis=0)
                    es.append(jax.lax.erf(zpair.astype(jnp.bfloat16)))
                    zv = znext + ds8
                acc = ((es[0] + es[1]).astype(jnp.float32)
                       + (es[2] + es[3]).astype(jnp.float32)
                       + (es[4] + es[5]).astype(jnp.float32)
                       + (es[6] + es[7]).astype(jnp.float32))
                t_rows.append(jnp.sum(acc, axis=0, keepdims=True))
            po_scr[pl.ds(r8, 1), :, lo:hi] = jnp.concatenate(
                t_rows, axis=0).reshape(1, 8, _C_BLK)
            return carry

        jax.lax.fori_loop(0, _NOCT, oct_body, 0, unroll=2)

    w_row = jnp.exp((-2.0 * _LN_S1) * t)
    acc2 = jnp.zeros((_B_BLK, _C_BLK), jnp.float32)
    for c in range(D // _C_BLK):
        lo, hi = c * _C_BLK, (c + 1) * _C_BLK
        bigt = po_scr[:, :, lo:hi].reshape(_B_BLK, _C_BLK)
        z0 = z0_scr[:, :, lo:hi].reshape(_B_BLK, _C_BLK)
        ds = ds_scr[:, :, lo:hi].reshape(_B_BLK, _C_BLK)
        e0 = jax.lax.erf(z0)
        e127 = jax.lax.erf(z0 + 127.0 * ds)
        pO = ((-127.0 / 256.0)
              + (1.0 / 128.0) * (e0 - bigt)
              + (127.0 / 256.0) * e127)
        d = x_ref[:, lo:hi] - pO
        acc2 = acc2 + d * d
    out_ref[...] = jnp.sum(w_row * acc2, axis=0, keepdims=True).reshape(
        1, 1, _C_BLK)


def kernel(x, t, noise, W1, b1, W2, b2):
    B, D = x.shape
    H = W1.shape[1]
    nb = B // _B_BLK
    grid = (nb,)
    parts = pl.pallas_call(
        _fused_kernel,
        grid=grid,
        in_specs=[
            pl.BlockSpec((_B_BLK, D), lambda i: (i, 0)),
            pl.BlockSpec((_B_BLK, 1), lambda i: (i, 0)),
            pl.BlockSpec((_B_BLK, D), lambda i: (i, 0)),
            pl.BlockSpec((D + 1, H), lambda i: (0, 0)),
            pl.BlockSpec((1, H), lambda i: (0, 0)),
            pl.BlockSpec((H, 2 * D), lambda i: (0, 0)),
            pl.BlockSpec((1, 2 * D), lambda i: (0, 0)),
        ],
        out_specs=pl.BlockSpec((1, 1, _C_BLK), lambda i: (i, 0, 0)),
        out_shape=jax.ShapeDtypeStruct((nb, 1, _C_BLK), jnp.float32),
        scratch_shapes=[
            pltpu.VMEM((_B_BLK, D), jnp.float32),
            pltpu.VMEM((_B_BLK, H), jnp.bfloat16),
            pltpu.VMEM((_B_BLK, 2 * D), jnp.float32),
            pltpu.VMEM((_NOCT, 8, D), jnp.float32),
            pltpu.VMEM((_NOCT, 8, D), jnp.float32),
            pltpu.VMEM((_NOCT, 8, D), jnp.float32),
        ],
        compiler_params=pltpu.CompilerParams(
            dimension_semantics=("arbitrary",),
            vmem_limit_bytes=100 * 1024 * 1024,
        ),
    )(x, t, noise, W1, b1.reshape(1, H), W2, b2.reshape(1, 2 * D))
    return (-_LN_S1 / (B * D)) * jnp.sum(parts)


# speedup vs baseline: 1.1019x; 1.0213x over previous
import numpy as np
import jax
import jax.numpy as jnp
from jax.experimental import pallas as pl
from jax.experimental.pallas import tpu as pltpu

_SIGMA1 = 0.02
_K = 128
_TMIN = 1e-10
_LEAKY = 0.01
_LN_S1 = float(np.log(_SIGMA1))

_B_BLK = 128
_C_BLK = 128
_NOCT = _B_BLK // 8


def _fused_kernel(x_ref, t_ref, noise_ref, W1_ref, b1_ref, W2_ref, b2_ref,
                  out_ref, mu_scr, h_scr, o_scr, z0_scr, ds_scr, po_scr):
    D = x_ref.shape[1]
    t = t_ref[...]
    gamma = 1.0 - jnp.exp((2.0 * _LN_S1) * t)
    mu_coef = gamma * (1.0 - gamma)
    mu_scr[...] = gamma * x_ref[...] + mu_coef * noise_ref[...]

    h = jnp.dot(mu_scr[...], W1_ref[:D, :], preferred_element_type=jnp.float32)
    h = h + t * W1_ref[D:D + 1, :] + b1_ref[...]
    h_scr[...] = jnp.where(h >= 0, h, _LEAKY * h).astype(jnp.bfloat16)

    o_scr[...] = jnp.dot(h_scr[...], W2_ref[...], preferred_element_type=jnp.float32)

    inv_gamma = 1.0 / gamma
    r = jnp.sqrt((1.0 - gamma) * inv_gamma)
    low_t = t < _TMIN

    for c in range(D // _C_BLK):
        lo, hi = c * _C_BLK, (c + 1) * _C_BLK
        mu_eps = o_scr[:, lo:hi] + b2_ref[:, lo:hi]
        ln_sig = o_scr[:, D + lo:D + hi] + b2_ref[:, D + lo:D + hi]
        mu_x = mu_scr[:, lo:hi] * inv_gamma - r * mu_eps
        sigma_x = r * jnp.exp(ln_sig)
        mu_x = jnp.where(low_t, 0.0, mu_x)
        sigma_x = jnp.where(low_t, 1.0, sigma_x)
        inv = jnp.minimum(1.0 / (sigma_x * jnp.float32(np.sqrt(2.0))), 1e30)
        ds_scr[:, :, lo:hi] = ((2.0 / _K) * inv).reshape(_NOCT, 8, _C_BLK)
        z0_scr[:, :, lo:hi] = ((-1.0 - mu_x) * inv).reshape(_NOCT, 8, _C_BLK)

    sub_iota = jax.lax.broadcasted_iota(
        jnp.int32, (8, _C_BLK), 0).astype(jnp.float32)
    for c in range(D // _C_BLK):
        lo, hi = c * _C_BLK, (c + 1) * _C_BLK

        def oct_body(r8, carry):
            z0o = z0_scr[pl.ds(r8, 1), :, lo:hi].reshape(8, _C_BLK)
            dso = ds_scr[pl.ds(r8, 1), :, lo:hi].reshape(8, _C_BLK)
            t_rows = []
            for rr in range(8):
                z0b = jnp.broadcast_to(z0o[rr:rr + 1, :], (8, _C_BLK))
                dsb = jnp.broadcast_to(dso[rr:rr + 1, :], (8, _C_BLK))
                ds8 = 8.0 * dsb
                zv = sub_iota * dsb + z0b
                es = []
                for v in range(8):
                    znext = zv + ds8
                    zpair = jnp.concatenate([zv, znext], axis=0)
                    es.append(jax.lax.erf(zpair.astype(jnp.bfloat16)))
                    zv = znext + ds8
                acc = ((es[0] + es[1]).astype(jnp.float32)
                       + (es[2] + es[3]).astype(jnp.float32)
                       + (es[4] + es[5]).astype(jnp.float32)
                       + (es[6] + es[7]).astype(jnp.float32))
                t_rows.append(jnp.sum(acc, axis=0, keepdims=True))
            po_scr[pl.ds(r8, 1), :, lo:hi] = jnp.concatenate(
                t_rows, axis=0).reshape(1, 8, _C_BLK)
            return carry

        jax.lax.fori_loop(0, _NOCT, oct_body, 0, unroll=4)

    w_row = jnp.exp((-2.0 * _LN_S1) * t)
    acc2 = jnp.zeros((_B_BLK, _C_BLK), jnp.float32)
    for c in range(D // _C_BLK):
        lo, hi = c * _C_BLK, (c + 1) * _C_BLK
        bigt = po_scr[:, :, lo:hi].reshape(_B_BLK, _C_BLK)
        z0 = z0_scr[:, :, lo:hi].reshape(_B_BLK, _C_BLK)
        ds = ds_scr[:, :, lo:hi].reshape(_B_BLK, _C_BLK)
        e0 = jax.lax.erf(z0)
        e127 = jax.lax.erf(z0 + 127.0 * ds)
        pO = ((-127.0 / 256.0)
              + (1.0 / 128.0) * (e0 - bigt)
              + (127.0 / 256.0) * e127)
        d = x_ref[:, lo:hi] - pO
        acc2 = acc2 + d * d
    out_ref[...] = jnp.sum(w_row * acc2, axis=0, keepdims=True).reshape(
        1, 1, _C_BLK)


def kernel(x, t, noise, W1, b1, W2, b2):
    B, D = x.shape
    H = W1.shape[1]
    nb = B // _B_BLK
    grid = (nb,)
    parts = pl.pallas_call(
        _fused_kernel,
        grid=grid,
        in_specs=[
            pl.BlockSpec((_B_BLK, D), lambda i: (i, 0)),
            pl.BlockSpec((_B_BLK, 1), lambda i: (i, 0)),
            pl.BlockSpec((_B_BLK, D), lambda i: (i, 0)),
            pl.BlockSpec((D + 1, H), lambda i: (0, 0)),
            pl.BlockSpec((1, H), lambda i: (0, 0)),
            pl.BlockSpec((H, 2 * D), lambda i: (0, 0)),
            pl.BlockSpec((1, 2 * D), lambda i: (0, 0)),
        ],
        out_specs=pl.BlockSpec((1, 1, _C_BLK), lambda i: (i, 0, 0)),
        out_shape=jax.ShapeDtypeStruct((nb, 1, _C_BLK), jnp.float32),
        scratch_shapes=[
            pltpu.VMEM((_B_BLK, D), jnp.float32),
            pltpu.VMEM((_B_BLK, H), jnp.bfloat16),
            pltpu.VMEM((_B_BLK, 2 * D), jnp.float32),
            pltpu.VMEM((_NOCT, 8, D), jnp.float32),
            pltpu.VMEM((_NOCT, 8, D), jnp.float32),
            pltpu.VMEM((_NOCT, 8, D), jnp.float32),
        ],
        compiler_params=pltpu.CompilerParams(
            dimension_semantics=("arbitrary",),
            vmem_limit_bytes=100 * 1024 * 1024,
        ),
    )(x, t, noise, W1, b1.reshape(1, H), W2, b2.reshape(1, 2 * D))
    return (-_LN_S1 / (B * D)) * jnp.sum(parts)
